# baseline (speedup 1.0000x reference)
import functools

import jax
import jax.numpy as jnp
from jax import lax
from jax.experimental import pallas as pl
from jax.experimental.pallas import tpu as pltpu
from jax.experimental.pallas import tpu_sc as plsc

N = 100000
E = 1600000
H = 32

NS = 16
NC = 2
SUB = 80
NSUB = 5
K = SUB * NSUB
EPT = E // NS
CH = EPT // K
ACC_PT = 6256
ACC_N = NS * ACC_PT

_mesh = plsc.VectorSubcoreMesh(core_axis_name="c", subcore_axis_name="s",
                               num_cores=NC, num_subcores=NS)


def _zero_acc(zsrc, acc, sid):
    def zrow(i, _):
        zsrc[i] = jnp.zeros((16,), jnp.float32)
        return 0
    lax.fori_loop(0, K, zrow, 0)
    base = sid * ACC_PT
    for z in range(15):
        pltpu.sync_copy(zsrc, acc.at[pl.ds(base + z * K, K)])
    pltpu.sync_copy(zsrc.at[pl.ds(0, 256)],
                    acc.at[pl.ds(base + 15 * K, 256)])


def _dump_acc(acc, out, sid, out_base):
    src = sid * ACC_PT

    @pl.when(sid < NS - 1)
    def _():
        pltpu.sync_copy(acc.at[pl.ds(src, ACC_PT)],
                        out.at[pl.ds(out_base + src, ACC_PT)])

    @pl.when(sid == NS - 1)
    def _():
        pltpu.sync_copy(acc.at[pl.ds(src, N - (NS - 1) * ACC_PT)],
                        out.at[pl.ds(out_base + src,
                                     N - (NS - 1) * ACC_PT)])


def _add_offset(idx, off):
    for q in range(K // 16):
        sl = pl.ds(q * 16, 16)
        idx[sl] = idx[sl] + off


@functools.partial(
    pl.kernel,
    out_type=jax.ShapeDtypeStruct((2 * N, 16), jnp.float32),
    mesh=_mesh,
    compiler_params=pltpu.CompilerParams(use_tc_tiling_on_sc=False),
    scratch_types=[
        pltpu.VMEM((K,), jnp.int32),
        pltpu.VMEM((K,), jnp.int32),
        pltpu.VMEM((K,), jnp.int32),
        pltpu.VMEM((K, 16), jnp.float32),
        pltpu.VMEM((K, 16), jnp.float32),
        pltpu.VMEM((K, 16), jnp.float32),
        pltpu.VMEM((K, 16), jnp.float32),
        pltpu.VMEM_SHARED((ACC_N, 16), jnp.float32),
        pltpu.SemaphoreType.DMA,
        pltpu.SemaphoreType.DMA,
        pltpu.SemaphoreType.DMA,
    ],
)
def _sc_layer(tbl, ef, srcA, dstA, out, srcv, dstv0, dstv1, rows, ev,
              mv0, mv1, acc, sem_g, sem_s0, sem_s1):
    cid = lax.axis_index("c")
    sid = lax.axis_index("s")
    _zero_acc(rows, acc, sid)
    plsc.subcore_barrier()

    def chunk(c, dstv, mv, sem_s):
        @pl.when(c >= 2)
        def _():
            for j in range(NSUB):
                pltpu.make_async_copy(
                    mv.at[pl.ds(j * SUB, SUB)],
                    acc.at[dstv.at[pl.ds(j * SUB, SUB)]], sem_s).wait()
        base = sid * EPT + c * K
        di = pltpu.async_copy(srcA.at[pl.ds(base, K)], srcv, sem_g)
        dd = pltpu.async_copy(dstA.at[pl.ds(base, K)], dstv, sem_g)
        di.wait()
        dd.wait()
        _add_offset(srcv, cid * N)
        descs = [pltpu.async_copy(tbl.at[srcv.at[pl.ds(j * SUB, SUB)]],
                                  rows.at[pl.ds(j * SUB, SUB)], sem_g)
                 for j in range(NSUB)]
        descs.append(
            pltpu.async_copy(ef.at[pl.ds(cid * E + base, K)], ev, sem_g))
        for d in descs:
            d.wait()

        @plsc.parallel_loop(0, K, step=1, unroll=8)
        def _(k):
            mv[k] = rows[k] * ev[k]

        for j in range(NSUB):
            pltpu.async_copy(mv.at[pl.ds(j * SUB, SUB)],
                             acc.at[dstv.at[pl.ds(j * SUB, SUB)]],
                             sem_s, add=True)

    def pair(g, _):
        chunk(2 * g, dstv0, mv0, sem_s0)
        chunk(2 * g + 1, dstv1, mv1, sem_s1)
        return 0
    lax.fori_loop(0, CH // 2, pair, 0)
    for dstv, mv, sem_s in ((dstv0, mv0, sem_s0), (dstv1, mv1, sem_s1)):
        for j in range(NSUB):
            pltpu.make_async_copy(
                mv.at[pl.ds(j * SUB, SUB)],
                acc.at[dstv.at[pl.ds(j * SUB, SUB)]], sem_s).wait()
    plsc.subcore_barrier()
    _dump_acc(acc, out, sid, cid * N)


@functools.partial(
    pl.kernel,
    out_type=jax.ShapeDtypeStruct((N, 16), jnp.float32),
    mesh=_mesh,
    compiler_params=pltpu.CompilerParams(use_tc_tiling_on_sc=False),
    scratch_types=[
        pltpu.VMEM((K,), jnp.int32),
        pltpu.VMEM((K,), jnp.int32),
        pltpu.VMEM((SUB, 16), jnp.float32),
        pltpu.VMEM((K, 16), jnp.float32),
        pltpu.VMEM_SHARED((ACC_N, 16), jnp.float32),
        pltpu.SemaphoreType.DMA,
        pltpu.SemaphoreType.DMA,
    ],
)
def _sc_degree(dstA, out, dstv0, dstv1, ones, zbuf, acc, sem_s0, sem_s1):
    cid = lax.axis_index("c")
    sid = lax.axis_index("s")

    def orow(i, _):
        ones[i] = jnp.ones((16,), jnp.float32)
        return 0
    lax.fori_loop(0, SUB, orow, 0)
    _zero_acc(zbuf, acc, sid)
    plsc.subcore_barrier()

    def chunk(c, dstv, sem_s):
        @pl.when(c >= 2)
        def _():
            for j in range(NSUB):
                pltpu.make_async_copy(
                    ones, acc.at[dstv.at[pl.ds(j * SUB, SUB)]],
                    sem_s).wait()
        base = sid * EPT + c * K
        pltpu.sync_copy(dstA.at[pl.ds(base, K)], dstv)
        for j in range(NSUB):
            pltpu.async_copy(ones, acc.at[dstv.at[pl.ds(j * SUB, SUB)]],
                             sem_s, add=True)

    def pair(g, _):
        chunk(2 * g, dstv0, sem_s0)
        chunk(2 * g + 1, dstv1, sem_s1)
        return 0
    lax.fori_loop(0, CH // 2, pair, 0)
    for dstv, sem_s in ((dstv0, sem_s0), (dstv1, sem_s1)):
        for j in range(NSUB):
            pltpu.make_async_copy(
                ones, acc.at[dstv.at[pl.ds(j * SUB, SUB)]], sem_s).wait()
    plsc.subcore_barrier()

    @pl.when(cid == 0)
    def _():
        _dump_acc(acc, out, sid, 0)


@functools.partial(
    pl.kernel,
    out_type=jax.ShapeDtypeStruct((2 * E // 8, 128), jnp.float32),
    mesh=_mesh,
    compiler_params=pltpu.CompilerParams(use_tc_tiling_on_sc=False),
    scratch_types=[
        pltpu.VMEM((K,), jnp.int32),
        pltpu.VMEM((K,), jnp.int32),
        pltpu.VMEM((K, 16), jnp.float32),
        pltpu.VMEM((K, 16), jnp.float32),
        pltpu.VMEM((K // 8, 128), jnp.float32),
        pltpu.VMEM((K // 8, 128), jnp.float32),
        pltpu.SemaphoreType.DMA,
        pltpu.SemaphoreType.DMA,
        pltpu.SemaphoreType.DMA,
        pltpu.SemaphoreType.DMA,
    ],
)
def _sc_decode(u, v, srcA, dstA, out, srcv, dstv, urows, vrows,
               sv0, sv1, sem_i, sem_g, sem_w0, sem_w1):
    cid = lax.axis_index("c")
    sid = lax.axis_index("s")

    def chunk(c, sv, sem_w):
        base = sid * EPT + c * K
        orow = (cid * E + base) // 8

        @pl.when(c >= 2)
        def _():
            pltpu.make_async_copy(sv, out.at[pl.ds(orow, K // 8)],
                                  sem_w).wait()
        di = pltpu.async_copy(srcA.at[pl.ds(base, K)], srcv, sem_i)
        dd = pltpu.async_copy(dstA.at[pl.ds(base, K)], dstv, sem_i)
        di.wait()
        dd.wait()
        _add_offset(srcv, cid * N)
        _add_offset(dstv, cid * N)
        descs = [pltpu.async_copy(u.at[srcv.at[pl.ds(j * SUB, SUB)]],
                                  urows.at[pl.ds(j * SUB, SUB)], sem_g)
                 for j in range(NSUB)]
        descs += [pltpu.async_copy(v.at[dstv.at[pl.ds(j * SUB, SUB)]],
                                   vrows.at[pl.ds(j * SUB, SUB)], sem_g)
                  for j in range(NSUB)]
        for d in descs:
            d.wait()

        @plsc.parallel_loop(0, K // 8, step=1, unroll=4)
        def _(q):
            for r in range(8):
                k = q * 8 + r
                sv[q, pl.ds(r * 16, 16)] = urows[k] + vrows[k]

        pltpu.async_copy(sv, out.at[pl.ds(orow, K // 8)], sem_w)

    def pair(g, _):
        chunk(2 * g, sv0, sem_w0)
        chunk(2 * g + 1, sv1, sem_w1)
        return 0
    lax.fori_loop(0, CH // 2, pair, 0)
    for c, sv, sem_w in ((CH - 2, sv0, sem_w0), (CH - 1, sv1, sem_w1)):
        orow = (cid * E + sid * EPT + c * K) // 8
        pltpu.make_async_copy(sv, out.at[pl.ds(orow, K // 8)],
                              sem_w).wait()


BN = 2000
GN = N // BN
BWE = 16000
GWE = E // BWE
BWF = 16000
GWF = E // BWF


def _full(shape):
    ndim = len(shape)
    return pl.BlockSpec(shape, lambda *a: (0,) * ndim)


def _rb(x):
    return x.astype(jnp.bfloat16).astype(jnp.float32)


def _tc_split(edge_index):
    def body(ei, out_s, out_d):
        out_s[...] = ei[0]
        out_d[...] = ei[1]

    return pl.pallas_call(
        body,
        grid=(1,),
        in_specs=[pl.BlockSpec((2, E), lambda i: (0, 0))],
        out_specs=[pl.BlockSpec((E,), lambda i: (0,))] * 2,
        out_shape=[jax.ShapeDtypeStruct((E,), jnp.int32)] * 2,
    )(edge_index)


BN0 = 2000


def _tc_encode_nodes(C, F, W1, b1, W2, b2):
    def body(c_b, f_b, W1_b, b1_b, W2_b, b2_b, out):
        w1 = _rb(W1_b[...])
        hid = (_rb(c_b[...]) * w1[:, 0][None, :]
               + _rb(f_b[...]) * w1[:, 1][None, :])
        hid = jnp.maximum(hid + b1_b[...][None, :], 0.0)
        out[...] = (jnp.dot(hid, W2_b[...].T,
                            preferred_element_type=jnp.float32)
                    + b2_b[0])

    gn0 = N // BN0
    return pl.pallas_call(
        body,
        grid=(2, gn0),
        in_specs=[pl.BlockSpec((BN0, 1), lambda h, i: (i, 0)),
                  pl.BlockSpec((BN0, 1), lambda h, i: (i, 0)),
                  _full((32, 2)), _full((32,)),
                  pl.BlockSpec((16, 32), lambda h, i: (h, 0)),
                  pl.BlockSpec((1, 1, 16), lambda h, i: (h, 0, 0))],
        out_specs=pl.BlockSpec((BN0, 16), lambda h, i: (h * gn0 + i, 0)),
        out_shape=jax.ShapeDtypeStruct((2 * N, 16), jnp.float32),
    )(C, F, W1, b1, W2, b2)


def _tc_encode(xs, W1, b1, W2, b2, nrows, bw, gw):
    nin = len(xs)

    def body(*refs):
        xr = refs[:nin]
        W1_b, b1_b, W2_b, b2_b, out = refs[nin:]
        hid = b1_b[...]
        for ci, x in enumerate(xr):
            hid = hid + _rb(W1_b[:, ci:ci + 1]) * _rb(x[...])
        hid = jnp.maximum(hid, 0.0)
        res = lax.dot_general(hid, W2_b[...], (((0,), (1,)), ((), ())),
                              preferred_element_type=jnp.float32)
        out[...] = res + b2_b[0]

    in_specs = ([pl.BlockSpec((1, bw), lambda h, i: (0, i))] * nin
                + [_full((32, nin)), _full((32, 1)),
                   pl.BlockSpec((16, 32), lambda h, i: (h, 0)),
                   pl.BlockSpec((1, 1, 16), lambda h, i: (h, 0, 0))])
    return pl.pallas_call(
        body,
        grid=(2, gw),
        in_specs=in_specs,
        out_specs=pl.BlockSpec((bw, 16), lambda h, i: (h * gw + i, 0)),
        out_shape=jax.ShapeDtypeStruct((2 * nrows, 16), jnp.float32),
    )(*xs, W1, b1, W2, b2)


def _tc_post12(accf, deg, nef, selfin, c2n, c2s, bias, layer1_w=None,
               invdeg=None):
    first = layer1_w is not None

    def body(a0, a1, ne0, ne1, dg, sfin, c2n_b, c2s_b, bias_b, *rest):
        if first:
            c1n_b, c1s_b, b1c_b = rest[:3]
            hsrc_o, self_o, inv_o = rest[3:]
            inv = 1.0 / jnp.maximum(dg[:, 0:1], 1.0)
            inv_o[...] = inv
        else:
            inv_b, = rest[:1]
            hsrc_o, self_o = rest[1:]
            inv = inv_b[...]
        acc = jnp.concatenate([a0[...], a1[...]], axis=1)
        ne = jnp.concatenate([ne0[...], ne1[...]], axis=1)
        mean = acc * inv
        if first:
            neigh = jnp.dot(mean, c1n_b[...].T,
                            preferred_element_type=jnp.float32)
            h = jnp.maximum(
                jnp.dot(ne, c1s_b[...].T,
                        preferred_element_type=jnp.float32)
                + neigh + b1c_b[...][None, :], 0.0)
        else:
            h = jnp.maximum(sfin[...] + mean + bias_b[...][None, :], 0.0)
        c2n_half = c2n_b[...]
        hsrc_o[...] = (
            jnp.dot(h, c2n_half[:, :32].T,
                    preferred_element_type=jnp.float32)
            + jnp.dot(ne, c2n_half[:, 32:].T,
                      preferred_element_type=jnp.float32))
        self_o[...] = (
            jnp.dot(h, c2s_b[...][:, :32].T,
                    preferred_element_type=jnp.float32)
            + jnp.dot(ne, c2s_b[...][:, 32:].T,
                      preferred_element_type=jnp.float32))

    half0 = pl.BlockSpec((BN, 16), lambda h, i: (i, 0))
    half1 = pl.BlockSpec((BN, 16), lambda h, i: (GN + i, 0))
    in_specs = [half0, half1, half0, half1,
                pl.BlockSpec((BN, 16), lambda h, i: (i, 0)),
                pl.BlockSpec((BN, 32), lambda h, i: (i, 0)),
                pl.BlockSpec((16, 64), lambda h, i: (h, 0)),
                _full((32, 64)), _full((32,))]
    out_specs = [pl.BlockSpec((BN, 16), lambda h, i: (h * GN + i, 0)),
                 pl.BlockSpec((BN, 32), lambda h, i: (i, 0))]
    out_shape = [jax.ShapeDtypeStruct((2 * N, 16), jnp.float32),
                 jax.ShapeDtypeStruct((N, 32), jnp.float32)]
    if first:
        in_specs += [_full((32, 32)), _full((32, 32)), _full((32,))]
        out_specs.append(pl.BlockSpec((BN, 1), lambda h, i: (i, 0)))
        out_shape.append(jax.ShapeDtypeStruct((N, 1), jnp.float32))
        extra = layer1_w
    else:
        in_specs.append(pl.BlockSpec((BN, 1), lambda h, i: (i, 0)))
        extra = (invdeg,)
    return pl.pallas_call(
        body,
        grid=(2, GN),
        in_specs=in_specs,
        out_specs=out_specs,
        out_shape=out_shape,
    )(accf, accf, nef, nef, deg, selfin, c2n, c2s, bias, *extra)


def _tc_post3(accf, invdeg, self3in, bias, W9):
    def body(a0, a1, inv_b, sfin, bias_b, W9_b_, u_o, v_o):
        acc = jnp.concatenate([a0[...], a1[...]], axis=1)
        h = sfin[...] + acc * inv_b[...] + bias_b[...][None, :]
        w9 = W9_b_[...]
        u_o[...] = jnp.dot(h, w9[:, :32].T,
                           preferred_element_type=jnp.float32)
        v_o[...] = jnp.dot(h, w9[:, 32:].T,
                           preferred_element_type=jnp.float32)

    half0 = pl.BlockSpec((BN, 16), lambda h, i: (i, 0))
    half1 = pl.BlockSpec((BN, 16), lambda h, i: (GN + i, 0))
    return pl.pallas_call(
        body,
        grid=(2, GN),
        in_specs=[half0, half1,
                  pl.BlockSpec((BN, 1), lambda h, i: (i, 0)),
                  pl.BlockSpec((BN, 32), lambda h, i: (i, 0)),
                  _full((32,)),
                  pl.BlockSpec((16, 64), lambda h, i: (h, 0))],
        out_specs=[pl.BlockSpec((BN, 16), lambda h, i: (h * GN + i, 0))] * 2,
        out_shape=[jax.ShapeDtypeStruct((2 * N, 16), jnp.float32)] * 2,
    )(accf, accf, invdeg, self3in, bias, W9)


def _tc_final(sf, cst, b10):
    RB = BWF // 8

    def body(s0, s1, cst_b, b10_b, p_o):
        cw = cst_b[...]
        t = (_rb(jnp.maximum(s0[...] + cw[0:1], 0.0)) * _rb(cw[2:3])
             + _rb(jnp.maximum(s1[...] + cw[1:2], 0.0)) * _rb(cw[3:4]))
        lane = lax.broadcasted_iota(jnp.int32, (128, 8), 0)
        col = lax.broadcasted_iota(jnp.int32, (128, 8), 1)
        m = (lane // 16 == col).astype(jnp.float32)
        g = jnp.dot(t, m, preferred_element_type=jnp.float32,
                    precision=lax.Precision.HIGHEST)
        p_o[...] = jnp.abs(g + b10_b[0])

    half0 = pl.BlockSpec((RB, 128), lambda i: (i, 0))
    half1 = pl.BlockSpec((RB, 128), lambda i: (E // 8 // RB + i, 0))
    return pl.pallas_call(
        body,
        grid=(GWF,),
        in_specs=[half0, half1, _full((4, 128)), _full((1,))],
        out_specs=pl.BlockSpec((RB, 8), lambda i: (i, 0)),
        out_shape=jax.ShapeDtypeStruct((E // 8, 8), jnp.float32),
    )(sf, sf, cst, b10)


def kernel(C, F, A, SP1, SP0, edge_index, W1_w, W1_b, W2_w, W2_b, W5_w, W5_b,
           W6_w, W6_b, conv1_self_w, conv1_neigh_w, conv1_bias, conv2_self_w,
           conv2_neigh_w, conv2_bias, W9_w, W9_b, W10_w, W10_b):
    src1d, dst1d = _tc_split(edge_index)

    nef = _tc_encode_nodes(C, F, W1_w, W1_b, W2_w,
                           W2_b.reshape(2, 1, 16))
    eef = _tc_encode([A.reshape(1, E), SP1.reshape(1, E),
                      SP0.reshape(1, E)],
                     W5_w, W5_b[:, None], W6_w, W6_b.reshape(2, 1, 16),
                     E, BWE, GWE)

    deg = _sc_degree(dst1d)
    acc1 = _sc_layer(nef, eef, src1d, dst1d)
    self_dummy = jnp.zeros((N, 32), jnp.float32)
    hsrc2, self2, invdeg = _tc_post12(
        acc1, deg, nef, self_dummy, conv2_neigh_w, conv2_self_w,
        conv2_bias, layer1_w=(conv1_neigh_w, conv1_self_w, conv1_bias))
    acc2 = _sc_layer(hsrc2, eef, src1d, dst1d)
    hsrc3, self3 = _tc_post12(acc2, deg, nef, self2, conv2_neigh_w,
                              conv2_self_w, conv2_bias, invdeg=invdeg)
    acc3 = _sc_layer(hsrc3, eef, src1d, dst1d)
    u, v = _tc_post3(acc3, invdeg, self3, conv2_bias, W9_w)

    sf = _sc_decode(u, v, src1d, dst1d)
    cst = jnp.stack([jnp.tile(W9_b[:16], 8), jnp.tile(W9_b[16:], 8),
                     jnp.tile(W10_w[0, :16], 8),
                     jnp.tile(W10_w[0, 16:], 8)])
    return _tc_final(sf, cst, W10_b).reshape(E)


# speedup vs baseline: 1.0718x; 1.0718x over previous
import functools

import jax
import jax.numpy as jnp
from jax import lax
from jax.experimental import pallas as pl
from jax.experimental.pallas import tpu as pltpu
from jax.experimental.pallas import tpu_sc as plsc

N = 100000
E = 1600000
H = 32

NS = 16
NC = 2
SUB = 80
NSUB = 5
K = SUB * NSUB
EPT = E // NS
CH = EPT // K
ACC_PT = 6256
ACC_N = NS * ACC_PT

_mesh = plsc.VectorSubcoreMesh(core_axis_name="c", subcore_axis_name="s",
                               num_cores=NC, num_subcores=NS)


def _zero_acc(zsrc, acc, sid):
    def zrow(i, _):
        zsrc[i] = jnp.zeros((16,), jnp.float32)
        return 0
    lax.fori_loop(0, K, zrow, 0)
    base = sid * ACC_PT
    for z in range(15):
        pltpu.sync_copy(zsrc, acc.at[pl.ds(base + z * K, K)])
    pltpu.sync_copy(zsrc.at[pl.ds(0, 256)],
                    acc.at[pl.ds(base + 15 * K, 256)])


def _dump_acc(acc, out, sid, out_base):
    src = sid * ACC_PT

    @pl.when(sid < NS - 1)
    def _():
        pltpu.sync_copy(acc.at[pl.ds(src, ACC_PT)],
                        out.at[pl.ds(out_base + src, ACC_PT)])

    @pl.when(sid == NS - 1)
    def _():
        pltpu.sync_copy(acc.at[pl.ds(src, N - (NS - 1) * ACC_PT)],
                        out.at[pl.ds(out_base + src,
                                     N - (NS - 1) * ACC_PT)])


def _add_offset(idx, off):
    for q in range(K // 16):
        sl = pl.ds(q * 16, 16)
        idx[sl] = idx[sl] + off


@functools.partial(
    pl.kernel,
    out_type=jax.ShapeDtypeStruct((2 * N, 16), jnp.float32),
    mesh=_mesh,
    compiler_params=pltpu.CompilerParams(use_tc_tiling_on_sc=False),
    scratch_types=[
        pltpu.VMEM((2, K), jnp.int32),
        pltpu.VMEM((4, K), jnp.int32),
        pltpu.VMEM((K, 16), jnp.float32),
        pltpu.VMEM((K, 16), jnp.float32),
        pltpu.VMEM((K, 16), jnp.float32),
        pltpu.VMEM((K, 16), jnp.float32),
        pltpu.VMEM_SHARED((ACC_N, 16), jnp.float32),
        pltpu.SemaphoreType.DMA,
        pltpu.SemaphoreType.DMA,
        pltpu.SemaphoreType.DMA,
        pltpu.SemaphoreType.DMA,
    ],
)
def _sc_layer(tbl, ef, srcA, dstA, out, srcv, dstv, rows, ev,
              mv0, mv1, acc, sem_i, sem_g, sem_s0, sem_s1):
    cid = lax.axis_index("c")
    sid = lax.axis_index("s")
    _zero_acc(rows, acc, sid)
    plsc.subcore_barrier()
    mvs = (mv0, mv1)
    sems = (sem_s0, sem_s1)

    def fire_idx(c, t):
        b = sid * EPT + c * K
        pltpu.async_copy(srcA.at[pl.ds(b, K)], srcv.at[t % 2], sem_i)
        pltpu.async_copy(dstA.at[pl.ds(b, K)], dstv.at[t % 4], sem_i)

    def chunk(c, t, fire_next=True):
        s2, s4 = t % 2, t % 4
        mv, sem_s = mvs[s2], sems[s2]
        dvp = dstv.at[(t + 2) % 4]
        base = sid * EPT + c * K

        @pl.when(c >= 2)
        def _():
            for j in range(NSUB):
                pltpu.make_async_copy(
                    mv.at[pl.ds(j * SUB, SUB)],
                    acc.at[dvp.at[pl.ds(j * SUB, SUB)]], sem_s).wait()
        pltpu.make_async_copy(srcA.at[pl.ds(base, K)], srcv.at[s2],
                              sem_i).wait()
        pltpu.make_async_copy(dstA.at[pl.ds(base, K)], dstv.at[s4],
                              sem_i).wait()
        if fire_next:
            fire_idx(c + 1, t + 1)
        sv = srcv.at[s2]
        _add_offset(sv, cid * N)
        descs = [pltpu.async_copy(tbl.at[sv.at[pl.ds(j * SUB, SUB)]],
                                  rows.at[pl.ds(j * SUB, SUB)], sem_g)
                 for j in range(NSUB)]
        descs.append(
            pltpu.async_copy(ef.at[pl.ds(cid * E + base, K)], ev, sem_g))
        for d in descs:
            d.wait()

        @plsc.parallel_loop(0, K, step=1, unroll=8)
        def _(k):
            mv[k] = rows[k] * ev[k]

        dv = dstv.at[s4]
        for j in range(NSUB):
            pltpu.async_copy(mv.at[pl.ds(j * SUB, SUB)],
                             acc.at[dv.at[pl.ds(j * SUB, SUB)]],
                             sem_s, add=True)

    fire_idx(jnp.int32(0), 0)

    def quad(g, _):
        for t in range(4):
            chunk(4 * g + t, t)
        return 0
    lax.fori_loop(0, (CH - 2) // 4, quad, 0)
    chunk(jnp.int32(CH - 2), 0)
    chunk(jnp.int32(CH - 1), 1, fire_next=False)
    for t in range(2):
        mv, sem_s = mvs[t], sems[t]
        dv = dstv.at[t]
        for j in range(NSUB):
            pltpu.make_async_copy(
                mv.at[pl.ds(j * SUB, SUB)],
                acc.at[dv.at[pl.ds(j * SUB, SUB)]], sem_s).wait()
    plsc.subcore_barrier()
    _dump_acc(acc, out, sid, cid * N)


@functools.partial(
    pl.kernel,
    out_type=jax.ShapeDtypeStruct((N, 16), jnp.float32),
    mesh=_mesh,
    compiler_params=pltpu.CompilerParams(use_tc_tiling_on_sc=False),
    scratch_types=[
        pltpu.VMEM((K,), jnp.int32),
        pltpu.VMEM((K,), jnp.int32),
        pltpu.VMEM((SUB, 16), jnp.float32),
        pltpu.VMEM((K, 16), jnp.float32),
        pltpu.VMEM_SHARED((ACC_N, 16), jnp.float32),
        pltpu.SemaphoreType.DMA,
        pltpu.SemaphoreType.DMA,
    ],
)
def _sc_degree(dstA, out, dstv0, dstv1, ones, zbuf, acc, sem_s0, sem_s1):
    cid = lax.axis_index("c")
    sid = lax.axis_index("s")

    def orow(i, _):
        ones[i] = jnp.ones((16,), jnp.float32)
        return 0
    lax.fori_loop(0, SUB, orow, 0)
    _zero_acc(zbuf, acc, sid)
    plsc.subcore_barrier()

    def chunk(c, dstv, sem_s):
        @pl.when(c >= 2)
        def _():
            for j in range(NSUB):
                pltpu.make_async_copy(
                    ones, acc.at[dstv.at[pl.ds(j * SUB, SUB)]],
                    sem_s).wait()
        base = sid * EPT + c * K
        pltpu.sync_copy(dstA.at[pl.ds(base, K)], dstv)
        for j in range(NSUB):
            pltpu.async_copy(ones, acc.at[dstv.at[pl.ds(j * SUB, SUB)]],
                             sem_s, add=True)

    def pair(g, _):
        chunk(2 * g, dstv0, sem_s0)
        chunk(2 * g + 1, dstv1, sem_s1)
        return 0
    lax.fori_loop(0, CH // 2, pair, 0)
    for dstv, sem_s in ((dstv0, sem_s0), (dstv1, sem_s1)):
        for j in range(NSUB):
            pltpu.make_async_copy(
                ones, acc.at[dstv.at[pl.ds(j * SUB, SUB)]], sem_s).wait()
    plsc.subcore_barrier()

    @pl.when(cid == 0)
    def _():
        _dump_acc(acc, out, sid, 0)


@functools.partial(
    pl.kernel,
    out_type=jax.ShapeDtypeStruct((2 * E // 8, 128), jnp.float32),
    mesh=_mesh,
    compiler_params=pltpu.CompilerParams(use_tc_tiling_on_sc=False),
    scratch_types=[
        pltpu.VMEM((K,), jnp.int32),
        pltpu.VMEM((K,), jnp.int32),
        pltpu.VMEM((K, 16), jnp.float32),
        pltpu.VMEM((K, 16), jnp.float32),
        pltpu.VMEM((K // 8, 128), jnp.float32),
        pltpu.VMEM((K // 8, 128), jnp.float32),
        pltpu.SemaphoreType.DMA,
        pltpu.SemaphoreType.DMA,
        pltpu.SemaphoreType.DMA,
        pltpu.SemaphoreType.DMA,
    ],
)
def _sc_decode(u, v, srcA, dstA, out, srcv, dstv, urows, vrows,
               sv0, sv1, sem_i, sem_g, sem_w0, sem_w1):
    cid = lax.axis_index("c")
    sid = lax.axis_index("s")

    def chunk(c, sv, sem_w):
        base = sid * EPT + c * K
        orow = (cid * E + base) // 8

        @pl.when(c >= 2)
        def _():
            pltpu.make_async_copy(sv, out.at[pl.ds(orow, K // 8)],
                                  sem_w).wait()
        di = pltpu.async_copy(srcA.at[pl.ds(base, K)], srcv, sem_i)
        dd = pltpu.async_copy(dstA.at[pl.ds(base, K)], dstv, sem_i)
        di.wait()
        dd.wait()
        _add_offset(srcv, cid * N)
        _add_offset(dstv, cid * N)
        descs = [pltpu.async_copy(u.at[srcv.at[pl.ds(j * SUB, SUB)]],
                                  urows.at[pl.ds(j * SUB, SUB)], sem_g)
                 for j in range(NSUB)]
        descs += [pltpu.async_copy(v.at[dstv.at[pl.ds(j * SUB, SUB)]],
                                   vrows.at[pl.ds(j * SUB, SUB)], sem_g)
                  for j in range(NSUB)]
        for d in descs:
            d.wait()

        @plsc.parallel_loop(0, K // 8, step=1, unroll=4)
        def _(q):
            for r in range(8):
                k = q * 8 + r
                sv[q, pl.ds(r * 16, 16)] = urows[k] + vrows[k]

        pltpu.async_copy(sv, out.at[pl.ds(orow, K // 8)], sem_w)

    def pair(g, _):
        chunk(2 * g, sv0, sem_w0)
        chunk(2 * g + 1, sv1, sem_w1)
        return 0
    lax.fori_loop(0, CH // 2, pair, 0)
    for c, sv, sem_w in ((CH - 2, sv0, sem_w0), (CH - 1, sv1, sem_w1)):
        orow = (cid * E + sid * EPT + c * K) // 8
        pltpu.make_async_copy(sv, out.at[pl.ds(orow, K // 8)],
                              sem_w).wait()


BN = 2000
GN = N // BN
BWE = 16000
GWE = E // BWE
BWF = 16000
GWF = E // BWF


def _full(shape):
    ndim = len(shape)
    return pl.BlockSpec(shape, lambda *a: (0,) * ndim)


def _rb(x):
    return x.astype(jnp.bfloat16).astype(jnp.float32)


def _tc_split(edge_index):
    def body(ei, out_s, out_d):
        out_s[...] = ei[0]
        out_d[...] = ei[1]

    return pl.pallas_call(
        body,
        grid=(1,),
        in_specs=[pl.BlockSpec((2, E), lambda i: (0, 0))],
        out_specs=[pl.BlockSpec((E,), lambda i: (0,))] * 2,
        out_shape=[jax.ShapeDtypeStruct((E,), jnp.int32)] * 2,
    )(edge_index)


BN0 = 2000


def _tc_encode_nodes(C, F, W1, b1, W2, b2):
    def body(c_b, f_b, W1_b, b1_b, W2_b, b2_b, out):
        w1 = _rb(W1_b[...])
        hid = (_rb(c_b[...]) * w1[:, 0][None, :]
               + _rb(f_b[...]) * w1[:, 1][None, :])
        hid = jnp.maximum(hid + b1_b[...][None, :], 0.0)
        out[...] = (jnp.dot(hid, W2_b[...].T,
                            preferred_element_type=jnp.float32)
                    + b2_b[0])

    gn0 = N // BN0
    return pl.pallas_call(
        body,
        grid=(2, gn0),
        in_specs=[pl.BlockSpec((BN0, 1), lambda h, i: (i, 0)),
                  pl.BlockSpec((BN0, 1), lambda h, i: (i, 0)),
                  _full((32, 2)), _full((32,)),
                  pl.BlockSpec((16, 32), lambda h, i: (h, 0)),
                  pl.BlockSpec((1, 1, 16), lambda h, i: (h, 0, 0))],
        out_specs=pl.BlockSpec((BN0, 16), lambda h, i: (h * gn0 + i, 0)),
        out_shape=jax.ShapeDtypeStruct((2 * N, 16), jnp.float32),
    )(C, F, W1, b1, W2, b2)


def _tc_encode(xs, W1, b1, W2, b2, nrows, bw, gw):
    nin = len(xs)

    def body(*refs):
        xr = refs[:nin]
        W1_b, b1_b, W2_b, b2_b, out = refs[nin:]
        hid = b1_b[...]
        for ci, x in enumerate(xr):
            hid = hid + _rb(W1_b[:, ci:ci + 1]) * _rb(x[...])
        hid = jnp.maximum(hid, 0.0)
        res = lax.dot_general(hid, W2_b[...], (((0,), (1,)), ((), ())),
                              preferred_element_type=jnp.float32)
        out[...] = res + b2_b[0]

    in_specs = ([pl.BlockSpec((1, bw), lambda h, i: (0, i))] * nin
                + [_full((32, nin)), _full((32, 1)),
                   pl.BlockSpec((16, 32), lambda h, i: (h, 0)),
                   pl.BlockSpec((1, 1, 16), lambda h, i: (h, 0, 0))])
    return pl.pallas_call(
        body,
        grid=(2, gw),
        in_specs=in_specs,
        out_specs=pl.BlockSpec((bw, 16), lambda h, i: (h * gw + i, 0)),
        out_shape=jax.ShapeDtypeStruct((2 * nrows, 16), jnp.float32),
    )(*xs, W1, b1, W2, b2)


def _tc_post12(accf, deg, nef, selfin, c2n, c2s, bias, layer1_w=None,
               invdeg=None):
    first = layer1_w is not None

    def body(a0, a1, ne0, ne1, dg, sfin, c2n_b, c2s_b, bias_b, *rest):
        if first:
            c1n_b, c1s_b, b1c_b = rest[:3]
            hsrc_o, self_o, inv_o = rest[3:]
            inv = 1.0 / jnp.maximum(dg[:, 0:1], 1.0)
            inv_o[...] = inv
        else:
            inv_b, = rest[:1]
            hsrc_o, self_o = rest[1:]
            inv = inv_b[...]
        acc = jnp.concatenate([a0[...], a1[...]], axis=1)
        ne = jnp.concatenate([ne0[...], ne1[...]], axis=1)
        mean = acc * inv
        if first:
            neigh = jnp.dot(mean, c1n_b[...].T,
                            preferred_element_type=jnp.float32)
            h = jnp.maximum(
                jnp.dot(ne, c1s_b[...].T,
                        preferred_element_type=jnp.float32)
                + neigh + b1c_b[...][None, :], 0.0)
        else:
            h = jnp.maximum(sfin[...] + mean + bias_b[...][None, :], 0.0)
        c2n_half = c2n_b[...]
        hsrc_o[...] = (
            jnp.dot(h, c2n_half[:, :32].T,
                    preferred_element_type=jnp.float32)
            + jnp.dot(ne, c2n_half[:, 32:].T,
                      preferred_element_type=jnp.float32))
        self_o[...] = (
            jnp.dot(h, c2s_b[...][:, :32].T,
                    preferred_element_type=jnp.float32)
            + jnp.dot(ne, c2s_b[...][:, 32:].T,
                      preferred_element_type=jnp.float32))

    half0 = pl.BlockSpec((BN, 16), lambda h, i: (i, 0))
    half1 = pl.BlockSpec((BN, 16), lambda h, i: (GN + i, 0))
    in_specs = [half0, half1, half0, half1,
                pl.BlockSpec((BN, 16), lambda h, i: (i, 0)),
                pl.BlockSpec((BN, 32), lambda h, i: (i, 0)),
                pl.BlockSpec((16, 64), lambda h, i: (h, 0)),
                _full((32, 64)), _full((32,))]
    out_specs = [pl.BlockSpec((BN, 16), lambda h, i: (h * GN + i, 0)),
                 pl.BlockSpec((BN, 32), lambda h, i: (i, 0))]
    out_shape = [jax.ShapeDtypeStruct((2 * N, 16), jnp.float32),
                 jax.ShapeDtypeStruct((N, 32), jnp.float32)]
    if first:
        in_specs += [_full((32, 32)), _full((32, 32)), _full((32,))]
        out_specs.append(pl.BlockSpec((BN, 1), lambda h, i: (i, 0)))
        out_shape.append(jax.ShapeDtypeStruct((N, 1), jnp.float32))
        extra = layer1_w
    else:
        in_specs.append(pl.BlockSpec((BN, 1), lambda h, i: (i, 0)))
        extra = (invdeg,)
    return pl.pallas_call(
        body,
        grid=(2, GN),
        in_specs=in_specs,
        out_specs=out_specs,
        out_shape=out_shape,
    )(accf, accf, nef, nef, deg, selfin, c2n, c2s, bias, *extra)


def _tc_post3(accf, invdeg, self3in, bias, W9):
    def body(a0, a1, inv_b, sfin, bias_b, W9_b_, u_o, v_o):
        acc = jnp.concatenate([a0[...], a1[...]], axis=1)
        h = sfin[...] + acc * inv_b[...] + bias_b[...][None, :]
        w9 = W9_b_[...]
        u_o[...] = jnp.dot(h, w9[:, :32].T,
                           preferred_element_type=jnp.float32)
        v_o[...] = jnp.dot(h, w9[:, 32:].T,
                           preferred_element_type=jnp.float32)

    half0 = pl.BlockSpec((BN, 16), lambda h, i: (i, 0))
    half1 = pl.BlockSpec((BN, 16), lambda h, i: (GN + i, 0))
    return pl.pallas_call(
        body,
        grid=(2, GN),
        in_specs=[half0, half1,
                  pl.BlockSpec((BN, 1), lambda h, i: (i, 0)),
                  pl.BlockSpec((BN, 32), lambda h, i: (i, 0)),
                  _full((32,)),
                  pl.BlockSpec((16, 64), lambda h, i: (h, 0))],
        out_specs=[pl.BlockSpec((BN, 16), lambda h, i: (h * GN + i, 0))] * 2,
        out_shape=[jax.ShapeDtypeStruct((2 * N, 16), jnp.float32)] * 2,
    )(accf, accf, invdeg, self3in, bias, W9)


def _tc_final(sf, cst, b10):
    RB = BWF // 8

    def body(s0, s1, cst_b, b10_b, p_o):
        cw = cst_b[...]
        t = (_rb(jnp.maximum(s0[...] + cw[0:1], 0.0)) * _rb(cw[2:3])
             + _rb(jnp.maximum(s1[...] + cw[1:2], 0.0)) * _rb(cw[3:4]))
        lane = lax.broadcasted_iota(jnp.int32, (128, 8), 0)
        col = lax.broadcasted_iota(jnp.int32, (128, 8), 1)
        m = (lane // 16 == col).astype(jnp.float32)
        g = jnp.dot(t, m, preferred_element_type=jnp.float32,
                    precision=lax.Precision.HIGHEST)
        p_o[...] = jnp.abs(g + b10_b[0])

    half0 = pl.BlockSpec((RB, 128), lambda i: (i, 0))
    half1 = pl.BlockSpec((RB, 128), lambda i: (E // 8 // RB + i, 0))
    return pl.pallas_call(
        body,
        grid=(GWF,),
        in_specs=[half0, half1, _full((4, 128)), _full((1,))],
        out_specs=pl.BlockSpec((RB, 8), lambda i: (i, 0)),
        out_shape=jax.ShapeDtypeStruct((E // 8, 8), jnp.float32),
    )(sf, sf, cst, b10)


def kernel(C, F, A, SP1, SP0, edge_index, W1_w, W1_b, W2_w, W2_b, W5_w, W5_b,
           W6_w, W6_b, conv1_self_w, conv1_neigh_w, conv1_bias, conv2_self_w,
           conv2_neigh_w, conv2_bias, W9_w, W9_b, W10_w, W10_b):
    src1d, dst1d = _tc_split(edge_index)

    nef = _tc_encode_nodes(C, F, W1_w, W1_b, W2_w,
                           W2_b.reshape(2, 1, 16))
    eef = _tc_encode([A.reshape(1, E), SP1.reshape(1, E),
                      SP0.reshape(1, E)],
                     W5_w, W5_b[:, None], W6_w, W6_b.reshape(2, 1, 16),
                     E, BWE, GWE)

    deg = _sc_degree(dst1d)
    acc1 = _sc_layer(nef, eef, src1d, dst1d)
    self_dummy = jnp.zeros((N, 32), jnp.float32)
    hsrc2, self2, invdeg = _tc_post12(
        acc1, deg, nef, self_dummy, conv2_neigh_w, conv2_self_w,
        conv2_bias, layer1_w=(conv1_neigh_w, conv1_self_w, conv1_bias))
    acc2 = _sc_layer(hsrc2, eef, src1d, dst1d)
    hsrc3, self3 = _tc_post12(acc2, deg, nef, self2, conv2_neigh_w,
                              conv2_self_w, conv2_bias, invdeg=invdeg)
    acc3 = _sc_layer(hsrc3, eef, src1d, dst1d)
    u, v = _tc_post3(acc3, invdeg, self3, conv2_bias, W9_w)

    sf = _sc_decode(u, v, src1d, dst1d)
    cst = jnp.stack([jnp.tile(W9_b[:16], 8), jnp.tile(W9_b[16:], 8),
                     jnp.tile(W10_w[0, :16], 8),
                     jnp.tile(W10_w[0, 16:], 8)])
    return _tc_final(sf, cst, W10_b).reshape(E)


# speedup vs baseline: 1.0955x; 1.0221x over previous
import functools

import jax
import jax.numpy as jnp
from jax import lax
from jax.experimental import pallas as pl
from jax.experimental.pallas import tpu as pltpu
from jax.experimental.pallas import tpu_sc as plsc

N = 100000
E = 1600000
H = 32

NS = 16
NC = 2
SUB = 80
NSUB = 5
K = SUB * NSUB
EPT = E // NS
CH = EPT // K
ACC_PT = 6256
ACC_N = NS * ACC_PT

_mesh = plsc.VectorSubcoreMesh(core_axis_name="c", subcore_axis_name="s",
                               num_cores=NC, num_subcores=NS)


def _zero_acc(zsrc, acc, sid):
    def zrow(i, _):
        zsrc[i] = jnp.zeros((16,), jnp.float32)
        return 0
    lax.fori_loop(0, K, zrow, 0)
    base = sid * ACC_PT
    for z in range(15):
        pltpu.sync_copy(zsrc, acc.at[pl.ds(base + z * K, K)])
    pltpu.sync_copy(zsrc.at[pl.ds(0, 256)],
                    acc.at[pl.ds(base + 15 * K, 256)])


def _dump_acc(acc, out, sid, out_base):
    src = sid * ACC_PT

    @pl.when(sid < NS - 1)
    def _():
        pltpu.sync_copy(acc.at[pl.ds(src, ACC_PT)],
                        out.at[pl.ds(out_base + src, ACC_PT)])

    @pl.when(sid == NS - 1)
    def _():
        pltpu.sync_copy(acc.at[pl.ds(src, N - (NS - 1) * ACC_PT)],
                        out.at[pl.ds(out_base + src,
                                     N - (NS - 1) * ACC_PT)])


def _add_offset(idx, off):
    for q in range(K // 16):
        sl = pl.ds(q * 16, 16)
        idx[sl] = idx[sl] + off


@functools.partial(
    pl.kernel,
    out_type=jax.ShapeDtypeStruct((2 * N, 16), jnp.float32),
    mesh=_mesh,
    compiler_params=pltpu.CompilerParams(use_tc_tiling_on_sc=False),
    scratch_types=[
        pltpu.VMEM((2, K), jnp.int32),
        pltpu.VMEM((4, K), jnp.int32),
        pltpu.VMEM((K, 16), jnp.float32),
        pltpu.VMEM((K, 16), jnp.float32),
        pltpu.VMEM((K, 16), jnp.float32),
        pltpu.VMEM((K, 16), jnp.float32),
        pltpu.VMEM_SHARED((ACC_N, 16), jnp.float32),
        pltpu.SemaphoreType.DMA,
        pltpu.SemaphoreType.DMA,
        pltpu.SemaphoreType.DMA,
        pltpu.SemaphoreType.DMA,
    ],
)
def _sc_layer(tbl, ef, srcA, dstA, out, srcv, dstv, rows, ev,
              mv0, mv1, acc, sem_i, sem_g, sem_s0, sem_s1):
    cid = lax.axis_index("c")
    sid = lax.axis_index("s")
    _zero_acc(rows, acc, sid)
    plsc.subcore_barrier()
    mvs = (mv0, mv1)
    sems = (sem_s0, sem_s1)

    def fire_idx(c, t):
        b = sid * EPT + c * K
        pltpu.async_copy(srcA.at[pl.ds(b, K)], srcv.at[t % 2], sem_i)
        pltpu.async_copy(dstA.at[pl.ds(b, K)], dstv.at[t % 4], sem_i)

    def chunk(c, t, fire_next=True):
        s2, s4 = t % 2, t % 4
        mv, sem_s = mvs[s2], sems[s2]
        dvp = dstv.at[(t + 2) % 4]
        base = sid * EPT + c * K

        @pl.when(c >= 2)
        def _():
            for j in range(NSUB):
                pltpu.make_async_copy(
                    mv.at[pl.ds(j * SUB, SUB)],
                    acc.at[dvp.at[pl.ds(j * SUB, SUB)]], sem_s).wait()
        pltpu.make_async_copy(srcA.at[pl.ds(base, K)], srcv.at[s2],
                              sem_i).wait()
        pltpu.make_async_copy(dstA.at[pl.ds(base, K)], dstv.at[s4],
                              sem_i).wait()
        if fire_next:
            fire_idx(c + 1, t + 1)
        sv = srcv.at[s2]
        _add_offset(sv, cid * N)
        descs = [pltpu.async_copy(tbl.at[sv.at[pl.ds(j * SUB, SUB)]],
                                  rows.at[pl.ds(j * SUB, SUB)], sem_g)
                 for j in range(NSUB)]
        descs.append(
            pltpu.async_copy(ef.at[pl.ds(cid * E + base, K)], ev, sem_g))
        for d in descs:
            d.wait()

        @plsc.parallel_loop(0, K, step=1, unroll=8)
        def _(k):
            mv[k] = rows[k] * ev[k]

        dv = dstv.at[s4]
        for j in range(NSUB):
            pltpu.async_copy(mv.at[pl.ds(j * SUB, SUB)],
                             acc.at[dv.at[pl.ds(j * SUB, SUB)]],
                             sem_s, add=True)

    fire_idx(jnp.int32(0), 0)

    def quad(g, _):
        for t in range(4):
            chunk(4 * g + t, t)
        return 0
    lax.fori_loop(0, (CH - 2) // 4, quad, 0)
    chunk(jnp.int32(CH - 2), 0)
    chunk(jnp.int32(CH - 1), 1, fire_next=False)
    for t in range(2):
        mv, sem_s = mvs[t], sems[t]
        dv = dstv.at[t]
        for j in range(NSUB):
            pltpu.make_async_copy(
                mv.at[pl.ds(j * SUB, SUB)],
                acc.at[dv.at[pl.ds(j * SUB, SUB)]], sem_s).wait()
    plsc.subcore_barrier()
    _dump_acc(acc, out, sid, cid * N)


@functools.partial(
    pl.kernel,
    out_type=jax.ShapeDtypeStruct((N, 16), jnp.float32),
    mesh=_mesh,
    compiler_params=pltpu.CompilerParams(use_tc_tiling_on_sc=False),
    scratch_types=[
        pltpu.VMEM((K,), jnp.int32),
        pltpu.VMEM((K,), jnp.int32),
        pltpu.VMEM((SUB, 16), jnp.float32),
        pltpu.VMEM((K, 16), jnp.float32),
        pltpu.VMEM_SHARED((ACC_N, 16), jnp.float32),
        pltpu.SemaphoreType.DMA,
        pltpu.SemaphoreType.DMA,
    ],
)
def _sc_degree(dstA, out, dstv0, dstv1, ones, zbuf, acc, sem_s0, sem_s1):
    cid = lax.axis_index("c")
    sid = lax.axis_index("s")

    def orow(i, _):
        ones[i] = jnp.ones((16,), jnp.float32)
        return 0
    lax.fori_loop(0, SUB, orow, 0)
    _zero_acc(zbuf, acc, sid)
    plsc.subcore_barrier()

    def chunk(c, dstv, sem_s):
        @pl.when(c >= 2)
        def _():
            for j in range(NSUB):
                pltpu.make_async_copy(
                    ones, acc.at[dstv.at[pl.ds(j * SUB, SUB)]],
                    sem_s).wait()
        base = sid * EPT + c * K
        pltpu.sync_copy(dstA.at[pl.ds(base, K)], dstv)
        for j in range(NSUB):
            pltpu.async_copy(ones, acc.at[dstv.at[pl.ds(j * SUB, SUB)]],
                             sem_s, add=True)

    def pair(g, _):
        chunk(2 * g, dstv0, sem_s0)
        chunk(2 * g + 1, dstv1, sem_s1)
        return 0
    lax.fori_loop(0, CH // 2, pair, 0)
    for dstv, sem_s in ((dstv0, sem_s0), (dstv1, sem_s1)):
        for j in range(NSUB):
            pltpu.make_async_copy(
                ones, acc.at[dstv.at[pl.ds(j * SUB, SUB)]], sem_s).wait()
    plsc.subcore_barrier()

    @pl.when(cid == 0)
    def _():
        _dump_acc(acc, out, sid, 0)


@functools.partial(
    pl.kernel,
    out_type=jax.ShapeDtypeStruct((2 * E // 8, 128), jnp.float32),
    mesh=_mesh,
    compiler_params=pltpu.CompilerParams(use_tc_tiling_on_sc=False),
    scratch_types=[
        pltpu.VMEM((2, K), jnp.int32),
        pltpu.VMEM((2, K), jnp.int32),
        pltpu.VMEM((K, 16), jnp.float32),
        pltpu.VMEM((K, 16), jnp.float32),
        pltpu.VMEM((K // 8, 128), jnp.float32),
        pltpu.VMEM((K // 8, 128), jnp.float32),
        pltpu.SemaphoreType.DMA,
        pltpu.SemaphoreType.DMA,
        pltpu.SemaphoreType.DMA,
        pltpu.SemaphoreType.DMA,
    ],
)
def _sc_decode(u, v, srcA, dstA, out, srcv, dstv, urows, vrows,
               sv0, sv1, sem_i, sem_g, sem_w0, sem_w1):
    cid = lax.axis_index("c")
    sid = lax.axis_index("s")
    svs = (sv0, sv1)
    sems = (sem_w0, sem_w1)

    def fire_idx(c, t):
        b = sid * EPT + c * K
        pltpu.async_copy(srcA.at[pl.ds(b, K)], srcv.at[t % 2], sem_i)
        pltpu.async_copy(dstA.at[pl.ds(b, K)], dstv.at[t % 2], sem_i)

    def chunk(c, t, fire_next=True):
        s2 = t % 2
        sv, sem_w = svs[s2], sems[s2]
        base = sid * EPT + c * K
        orow = (cid * E + base) // 8

        @pl.when(c >= 2)
        def _():
            pltpu.make_async_copy(sv, out.at[pl.ds(orow, K // 8)],
                                  sem_w).wait()
        pltpu.make_async_copy(srcA.at[pl.ds(base, K)], srcv.at[s2],
                              sem_i).wait()
        pltpu.make_async_copy(dstA.at[pl.ds(base, K)], dstv.at[s2],
                              sem_i).wait()
        if fire_next:
            fire_idx(c + 1, t + 1)
        svi = srcv.at[s2]
        dvi = dstv.at[s2]
        _add_offset(svi, cid * N)
        _add_offset(dvi, cid * N)
        descs = [pltpu.async_copy(u.at[svi.at[pl.ds(j * SUB, SUB)]],
                                  urows.at[pl.ds(j * SUB, SUB)], sem_g)
                 for j in range(NSUB)]
        descs += [pltpu.async_copy(v.at[dvi.at[pl.ds(j * SUB, SUB)]],
                                   vrows.at[pl.ds(j * SUB, SUB)], sem_g)
                  for j in range(NSUB)]
        for d in descs:
            d.wait()

        @plsc.parallel_loop(0, K // 8, step=1, unroll=4)
        def _(q):
            for r in range(8):
                k = q * 8 + r
                sv[q, pl.ds(r * 16, 16)] = urows[k] + vrows[k]

        pltpu.async_copy(sv, out.at[pl.ds(orow, K // 8)], sem_w)

    fire_idx(jnp.int32(0), 0)

    def pair(g, _):
        chunk(2 * g, 0)
        chunk(2 * g + 1, 1)
        return 0
    lax.fori_loop(0, (CH - 2) // 2, pair, 0)
    chunk(jnp.int32(CH - 2), 0)
    chunk(jnp.int32(CH - 1), 1, fire_next=False)
    for c, t in ((CH - 2, 0), (CH - 1, 1)):
        orow = (cid * E + sid * EPT + c * K) // 8
        pltpu.make_async_copy(svs[t], out.at[pl.ds(orow, K // 8)],
                              sems[t]).wait()


BN = 2000
GN = N // BN
BWE = 16000
GWE = E // BWE
BWF = 16000
GWF = E // BWF


def _full(shape):
    ndim = len(shape)
    return pl.BlockSpec(shape, lambda *a: (0,) * ndim)


def _rb(x):
    return x.astype(jnp.bfloat16).astype(jnp.float32)


def _tc_split(edge_index):
    def body(ei, out_s, out_d):
        out_s[...] = ei[0]
        out_d[...] = ei[1]

    return pl.pallas_call(
        body,
        grid=(1,),
        in_specs=[pl.BlockSpec((2, E), lambda i: (0, 0))],
        out_specs=[pl.BlockSpec((E,), lambda i: (0,))] * 2,
        out_shape=[jax.ShapeDtypeStruct((E,), jnp.int32)] * 2,
    )(edge_index)


BN0 = 2000


def _tc_encode_nodes(C, F, W1, b1, W2, b2):
    def body(c_b, f_b, W1_b, b1_b, W2_b, b2_b, out):
        w1 = _rb(W1_b[...])
        hid = (_rb(c_b[...]) * w1[:, 0][None, :]
               + _rb(f_b[...]) * w1[:, 1][None, :])
        hid = jnp.maximum(hid + b1_b[...][None, :], 0.0)
        out[...] = (jnp.dot(hid, W2_b[...].T,
                            preferred_element_type=jnp.float32)
                    + b2_b[0])

    gn0 = N // BN0
    return pl.pallas_call(
        body,
        grid=(2, gn0),
        in_specs=[pl.BlockSpec((BN0, 1), lambda h, i: (i, 0)),
                  pl.BlockSpec((BN0, 1), lambda h, i: (i, 0)),
                  _full((32, 2)), _full((32,)),
                  pl.BlockSpec((16, 32), lambda h, i: (h, 0)),
                  pl.BlockSpec((1, 1, 16), lambda h, i: (h, 0, 0))],
        out_specs=pl.BlockSpec((BN0, 16), lambda h, i: (h * gn0 + i, 0)),
        out_shape=jax.ShapeDtypeStruct((2 * N, 16), jnp.float32),
    )(C, F, W1, b1, W2, b2)


def _tc_encode(xs, W1, b1, W2, b2, nrows, bw, gw):
    nin = len(xs)

    def body(*refs):
        xr = refs[:nin]
        W1_b, b1_b, W2_b, b2_b, out = refs[nin:]
        hid = b1_b[...]
        for ci, x in enumerate(xr):
            hid = hid + _rb(W1_b[:, ci:ci + 1]) * _rb(x[...])
        hid = jnp.maximum(hid, 0.0)
        res = lax.dot_general(hid, W2_b[...], (((0,), (1,)), ((), ())),
                              preferred_element_type=jnp.float32)
        out[...] = res + b2_b[0]

    in_specs = ([pl.BlockSpec((1, bw), lambda h, i: (0, i))] * nin
                + [_full((32, nin)), _full((32, 1)),
                   pl.BlockSpec((16, 32), lambda h, i: (h, 0)),
                   pl.BlockSpec((1, 1, 16), lambda h, i: (h, 0, 0))])
    return pl.pallas_call(
        body,
        grid=(2, gw),
        in_specs=in_specs,
        out_specs=pl.BlockSpec((bw, 16), lambda h, i: (h * gw + i, 0)),
        out_shape=jax.ShapeDtypeStruct((2 * nrows, 16), jnp.float32),
    )(*xs, W1, b1, W2, b2)


def _tc_post12(accf, deg, nef, selfin, c2n, c2s, bias, layer1_w=None,
               invdeg=None):
    first = layer1_w is not None

    def body(a0, a1, ne0, ne1, dg, sfin, c2n_b, c2s_b, bias_b, *rest):
        if first:
            c1n_b, c1s_b, b1c_b = rest[:3]
            hsrc_o, self_o, inv_o = rest[3:]
            inv = 1.0 / jnp.maximum(dg[:, 0:1], 1.0)
            inv_o[...] = inv
        else:
            inv_b, = rest[:1]
            hsrc_o, self_o = rest[1:]
            inv = inv_b[...]
        acc = jnp.concatenate([a0[...], a1[...]], axis=1)
        ne = jnp.concatenate([ne0[...], ne1[...]], axis=1)
        mean = acc * inv
        if first:
            neigh = jnp.dot(mean, c1n_b[...].T,
                            preferred_element_type=jnp.float32)
            h = jnp.maximum(
                jnp.dot(ne, c1s_b[...].T,
                        preferred_element_type=jnp.float32)
                + neigh + b1c_b[...][None, :], 0.0)
        else:
            h = jnp.maximum(sfin[...] + mean + bias_b[...][None, :], 0.0)
        c2n_half = c2n_b[...]
        hsrc_o[...] = (
            jnp.dot(h, c2n_half[:, :32].T,
                    preferred_element_type=jnp.float32)
            + jnp.dot(ne, c2n_half[:, 32:].T,
                      preferred_element_type=jnp.float32))
        self_o[...] = (
            jnp.dot(h, c2s_b[...][:, :32].T,
                    preferred_element_type=jnp.float32)
            + jnp.dot(ne, c2s_b[...][:, 32:].T,
                      preferred_element_type=jnp.float32))

    half0 = pl.BlockSpec((BN, 16), lambda h, i: (i, 0))
    half1 = pl.BlockSpec((BN, 16), lambda h, i: (GN + i, 0))
    in_specs = [half0, half1, half0, half1,
                pl.BlockSpec((BN, 16), lambda h, i: (i, 0)),
                pl.BlockSpec((BN, 32), lambda h, i: (i, 0)),
                pl.BlockSpec((16, 64), lambda h, i: (h, 0)),
                _full((32, 64)), _full((32,))]
    out_specs = [pl.BlockSpec((BN, 16), lambda h, i: (h * GN + i, 0)),
                 pl.BlockSpec((BN, 32), lambda h, i: (i, 0))]
    out_shape = [jax.ShapeDtypeStruct((2 * N, 16), jnp.float32),
                 jax.ShapeDtypeStruct((N, 32), jnp.float32)]
    if first:
        in_specs += [_full((32, 32)), _full((32, 32)), _full((32,))]
        out_specs.append(pl.BlockSpec((BN, 1), lambda h, i: (i, 0)))
        out_shape.append(jax.ShapeDtypeStruct((N, 1), jnp.float32))
        extra = layer1_w
    else:
        in_specs.append(pl.BlockSpec((BN, 1), lambda h, i: (i, 0)))
        extra = (invdeg,)
    return pl.pallas_call(
        body,
        grid=(2, GN),
        in_specs=in_specs,
        out_specs=out_specs,
        out_shape=out_shape,
    )(accf, accf, nef, nef, deg, selfin, c2n, c2s, bias, *extra)


def _tc_post3(accf, invdeg, self3in, bias, W9):
    def body(a0, a1, inv_b, sfin, bias_b, W9_b_, u_o, v_o):
        acc = jnp.concatenate([a0[...], a1[...]], axis=1)
        h = sfin[...] + acc * inv_b[...] + bias_b[...][None, :]
        w9 = W9_b_[...]
        u_o[...] = jnp.dot(h, w9[:, :32].T,
                           preferred_element_type=jnp.float32)
        v_o[...] = jnp.dot(h, w9[:, 32:].T,
                           preferred_element_type=jnp.float32)

    half0 = pl.BlockSpec((BN, 16), lambda h, i: (i, 0))
    half1 = pl.BlockSpec((BN, 16), lambda h, i: (GN + i, 0))
    return pl.pallas_call(
        body,
        grid=(2, GN),
        in_specs=[half0, half1,
                  pl.BlockSpec((BN, 1), lambda h, i: (i, 0)),
                  pl.BlockSpec((BN, 32), lambda h, i: (i, 0)),
                  _full((32,)),
                  pl.BlockSpec((16, 64), lambda h, i: (h, 0))],
        out_specs=[pl.BlockSpec((BN, 16), lambda h, i: (h * GN + i, 0))] * 2,
        out_shape=[jax.ShapeDtypeStruct((2 * N, 16), jnp.float32)] * 2,
    )(accf, accf, invdeg, self3in, bias, W9)


def _tc_final(sf, cst, b10):
    RB = BWF // 8

    def body(s0, s1, cst_b, b10_b, p_o):
        cw = cst_b[...]
        t = (_rb(jnp.maximum(s0[...] + cw[0:1], 0.0)) * _rb(cw[2:3])
             + _rb(jnp.maximum(s1[...] + cw[1:2], 0.0)) * _rb(cw[3:4]))
        lane = lax.broadcasted_iota(jnp.int32, (128, 8), 0)
        col = lax.broadcasted_iota(jnp.int32, (128, 8), 1)
        m = (lane // 16 == col).astype(jnp.float32)
        g = jnp.dot(t, m, preferred_element_type=jnp.float32,
                    precision=lax.Precision.HIGHEST)
        p_o[...] = jnp.abs(g + b10_b[0])

    half0 = pl.BlockSpec((RB, 128), lambda i: (i, 0))
    half1 = pl.BlockSpec((RB, 128), lambda i: (E // 8 // RB + i, 0))
    return pl.pallas_call(
        body,
        grid=(GWF,),
        in_specs=[half0, half1, _full((4, 128)), _full((1,))],
        out_specs=pl.BlockSpec((RB, 8), lambda i: (i, 0)),
        out_shape=jax.ShapeDtypeStruct((E // 8, 8), jnp.float32),
    )(sf, sf, cst, b10)


def kernel(C, F, A, SP1, SP0, edge_index, W1_w, W1_b, W2_w, W2_b, W5_w, W5_b,
           W6_w, W6_b, conv1_self_w, conv1_neigh_w, conv1_bias, conv2_self_w,
           conv2_neigh_w, conv2_bias, W9_w, W9_b, W10_w, W10_b):
    src1d, dst1d = _tc_split(edge_index)

    nef = _tc_encode_nodes(C, F, W1_w, W1_b, W2_w,
                           W2_b.reshape(2, 1, 16))
    eef = _tc_encode([A.reshape(1, E), SP1.reshape(1, E),
                      SP0.reshape(1, E)],
                     W5_w, W5_b[:, None], W6_w, W6_b.reshape(2, 1, 16),
                     E, BWE, GWE)

    deg = _sc_degree(dst1d)
    acc1 = _sc_layer(nef, eef, src1d, dst1d)
    self_dummy = jnp.zeros((N, 32), jnp.float32)
    hsrc2, self2, invdeg = _tc_post12(
        acc1, deg, nef, self_dummy, conv2_neigh_w, conv2_self_w,
        conv2_bias, layer1_w=(conv1_neigh_w, conv1_self_w, conv1_bias))
    acc2 = _sc_layer(hsrc2, eef, src1d, dst1d)
    hsrc3, self3 = _tc_post12(acc2, deg, nef, self2, conv2_neigh_w,
                              conv2_self_w, conv2_bias, invdeg=invdeg)
    acc3 = _sc_layer(hsrc3, eef, src1d, dst1d)
    u, v = _tc_post3(acc3, invdeg, self3, conv2_bias, W9_w)

    sf = _sc_decode(u, v, src1d, dst1d)
    cst = jnp.stack([jnp.tile(W9_b[:16], 8), jnp.tile(W9_b[16:], 8),
                     jnp.tile(W10_w[0, :16], 8),
                     jnp.tile(W10_w[0, 16:], 8)])
    return _tc_final(sf, cst, W10_b).reshape(E)
